# baseline (speedup 1.0000x reference)
import functools

import jax
import jax.numpy as jnp
from jax import lax
from jax.experimental import pallas as pl
from jax.experimental.pallas import tpu as pltpu
from jax.experimental.pallas import tpu_sc as plsc

_E = 8
_EPAD = 128
_BLK = 512
_NW = 32
_L = 16


def _logits_body(hs_ref, w_ref, out_ref):
    x = hs_ref[...]
    logits = jnp.dot(x, w_ref[...],
                     preferred_element_type=jnp.float32)
    out_ref[...] = logits.T


def _scale_sc_body(logits_hbm, scale_hbm, l_v, s_v):
    chunk = s_v.shape[0]
    wid = lax.axis_index("s") * 2 + lax.axis_index("c")
    base = wid * chunk
    pltpu.sync_copy(logits_hbm.at[:, pl.ds(base, chunk)], l_v)

    def body(i, carry):
        off = i * _L
        les = [l_v[e, pl.ds(off, _L)] for e in range(_E)]
        m1 = les[0]
        m2 = jnp.full((_L,), -jnp.inf, jnp.float32)
        for e in range(1, _E):
            m2 = jnp.maximum(m2, jnp.minimum(m1, les[e]))
            m1 = jnp.maximum(m1, les[e])
        sumexp = jnp.zeros((_L,), jnp.float32)
        for e in range(_E):
            sumexp = sumexp + jnp.exp(les[e] - m1)
        s_v[pl.ds(off, _L)] = (1.0 + jnp.exp(m2 - m1)) / sumexp
        return carry

    lax.fori_loop(0, chunk // _L, body, 0)
    pltpu.sync_copy(s_v, scale_hbm.at[pl.ds(base, chunk)])


def _mul_body(hs_ref, sc_ref, out_ref):
    out_ref[...] = hs_ref[...] * sc_ref[...]


@jax.jit
def kernel(hidden_states, W_router):
    B, S, H = hidden_states.shape
    E = W_router.shape[-1]
    rows = B * S
    blk = min(_BLK, rows)
    chunk = rows // _NW
    hs2d = hidden_states.reshape(rows, H)
    w_pad = jnp.zeros((H, _EPAD), dtype=W_router.dtype).at[:, :E].set(W_router)

    logits_t = pl.pallas_call(
        _logits_body,
        grid=(rows // blk,),
        in_specs=[
            pl.BlockSpec((blk, H), lambda i: (i, 0)),
            pl.BlockSpec((H, _EPAD), lambda i: (0, 0)),
        ],
        out_specs=pl.BlockSpec((_EPAD, blk), lambda i: (0, i)),
        out_shape=jax.ShapeDtypeStruct((_EPAD, rows), jnp.float32),
    )(hs2d, w_pad)
    logits_t = logits_t[:_E]

    mesh = plsc.VectorSubcoreMesh(core_axis_name="c", subcore_axis_name="s")
    scale = pl.kernel(
        _scale_sc_body,
        mesh=mesh,
        out_type=jax.ShapeDtypeStruct((rows,), jnp.float32),
        scratch_types=[
            pltpu.VMEM((_E, chunk), jnp.float32),
            pltpu.VMEM((chunk,), jnp.float32),
        ],
    )(logits_t)

    out = pl.pallas_call(
        _mul_body,
        grid=(rows // blk,),
        in_specs=[
            pl.BlockSpec((blk, H), lambda i: (i, 0)),
            pl.BlockSpec((blk, 1), lambda i: (i, 0)),
        ],
        out_specs=pl.BlockSpec((blk, H), lambda i: (i, 0)),
        out_shape=jax.ShapeDtypeStruct((rows, H), hidden_states.dtype),
    )(hs2d, scale.reshape(rows, 1))
    return out.reshape(B, S, H)


# speedup vs baseline: 1.6351x; 1.6351x over previous
import functools

import jax
import jax.numpy as jnp
from jax.experimental import pallas as pl
from jax.experimental.pallas import tpu as pltpu

_E = 8
_EPAD = 128
_BLK = 512


def _fused_body(hs_ref, w_ref, out_ref):
    x = hs_ref[...]
    logits = jnp.dot(x, w_ref[...],
                     preferred_element_type=jnp.float32)
    lane = jax.lax.broadcasted_iota(jnp.int32, logits.shape, 1)
    neg_inf = jnp.float32(float("-inf"))
    logits = jnp.where(lane < _E, logits, neg_inf)

    m1 = jnp.max(logits, axis=-1, keepdims=True)
    idx1 = jnp.min(jnp.where(logits == m1, lane, _EPAD), axis=-1,
                   keepdims=True)
    masked = jnp.where(lane == idx1, neg_inf, logits)
    m2 = jnp.max(masked, axis=-1, keepdims=True)

    sumexp = jnp.sum(jnp.exp(logits - m1), axis=-1, keepdims=True)
    scale = (1.0 + jnp.exp(m2 - m1)) / sumexp
    out_ref[...] = x * scale


@jax.jit
def kernel(hidden_states, W_router):
    B, S, H = hidden_states.shape
    E = W_router.shape[-1]
    rows = B * S
    blk = min(_BLK, rows)
    hs2d = hidden_states.reshape(rows, H)
    w_pad = jnp.zeros((H, _EPAD), dtype=W_router.dtype).at[:, :E].set(W_router)

    out = pl.pallas_call(
        _fused_body,
        grid=(rows // blk,),
        in_specs=[
            pl.BlockSpec((blk, H), lambda i: (i, 0)),
            pl.BlockSpec((H, _EPAD), lambda i: (0, 0)),
        ],
        out_specs=pl.BlockSpec((blk, H), lambda i: (i, 0)),
        out_shape=jax.ShapeDtypeStruct((rows, H), hidden_states.dtype),
        compiler_params=pltpu.CompilerParams(
            dimension_semantics=("parallel",),
        ),
    )(hs2d, w_pad)
    return out.reshape(B, S, H)


# speedup vs baseline: 1.6404x; 1.0033x over previous
import jax
import jax.numpy as jnp
from jax.experimental import pallas as pl
from jax.experimental.pallas import tpu as pltpu

_E = 8
_EPAD = 128
_BLK = 512


def _fused_body(hs_ref, w_ref, out_ref):
    x = hs_ref[...]
    logits = jnp.dot(x, w_ref[...],
                     preferred_element_type=jnp.float32)
    lane = jax.lax.broadcasted_iota(jnp.int32, logits.shape, 1)
    neg_inf = jnp.float32(float("-inf"))
    logits = jnp.where(lane < _E, logits, neg_inf)

    m1 = jnp.max(logits, axis=-1, keepdims=True)
    idx1 = jnp.min(jnp.where(logits == m1, lane, _EPAD), axis=-1,
                   keepdims=True)
    masked = jnp.where(lane == idx1, neg_inf, logits)
    m2 = jnp.max(masked, axis=-1, keepdims=True)

    sumexp = jnp.sum(jnp.exp(logits - m1), axis=-1, keepdims=True)
    scale = (1.0 + jnp.exp(m2 - m1)) / sumexp
    out_ref[...] = x * scale


@jax.jit
def kernel(hidden_states, W_router):
    B, S, H = hidden_states.shape
    E = W_router.shape[-1]
    rows = B * S
    blk = min(_BLK, rows)
    hs2d = hidden_states.reshape(rows, H)
    w_pad = jnp.zeros((H, _EPAD), dtype=W_router.dtype).at[:, :E].set(W_router)

    out = pl.pallas_call(
        _fused_body,
        grid=(rows // blk,),
        in_specs=[
            pl.BlockSpec((blk, H), lambda i: (i, 0)),
            pl.BlockSpec((H, _EPAD), lambda i: (0, 0)),
        ],
        out_specs=pl.BlockSpec((blk, H), lambda i: (i, 0)),
        out_shape=jax.ShapeDtypeStruct((rows, H), hidden_states.dtype),
    )(hs2d, w_pad)
    return out.reshape(B, S, H)
